# baseline (speedup 1.0000x reference)
import functools

import jax
import jax.numpy as jnp
from jax import lax
from jax.experimental import pallas as pl
from jax.experimental.pallas import tpu as pltpu
from jax.experimental.pallas import tpu_sc as plsc

N = 10000
E = 320000
D = 128
NC = 2
NS = 16
NW = NC * NS
C = 128
G = 4
JT = 2560
EP = C * JT
J0 = 156
J1 = 4
NP = 10240
RPT = NP // NS
CW = 128


def _mesh():
    return plsc.VectorSubcoreMesh(core_axis_name="c", subcore_axis_name="s",
                                  num_cores=NC, num_subcores=NS)


def _seg_body(h_hbm, srcs_hbm, dsts_hbm, acc_hbm,
              idx_src, idx_dst, rows, acc_sh,
              sem_g0, sem_g1, sem_s0, sem_s1, sem_is, sem_id):
    cid = lax.axis_index("c")
    sid = lax.axis_index("s")
    sem_g = (sem_g0, sem_g1)
    sem_s = (sem_s0, sem_s1)
    is0 = cid == 0
    start = jnp.where(is0, sid * J0, NS * J0 + sid * J1)
    NG = jnp.where(is0, J0 // G, J1 // G)

    zv = jnp.zeros((16,), jnp.float32)

    def _zrow(r, carry):
        for k in range(D // 16):
            rows[0, r, pl.ds(k * 16, 16)] = zv
        return carry
    lax.fori_loop(0, C, _zrow, 0)
    for k in range(RPT // C):
        pltpu.sync_copy(rows.at[0], acc_sh.at[pl.ds(sid * RPT + k * C, C)])

    pltpu.sync_copy(srcs_hbm.at[pl.ds(start, G)], idx_src.at[0])
    pltpu.sync_copy(dsts_hbm.at[pl.ds(start, G)], idx_dst.at[0])
    pltpu.async_copy(h_hbm.at[idx_src.at[0, 0]], rows.at[0], sem_g[0])

    plsc.subcore_barrier()

    def _group(g, carry):
        s_cur = g % 2
        s_nxt = (g + 1) % 2
        for jj in range(G):
            b = jj % 2
            pltpu.make_async_copy(h_hbm.at[idx_src.at[s_cur, jj]],
                                  rows.at[b], sem_g[b]).wait()
            if jj == 0:
                @pl.when(g >= 1)
                def _():
                    pltpu.make_async_copy(
                        rows.at[1 - b], acc_sh.at[idx_dst.at[s_cur, 0]],
                        sem_s[1 - b]).wait()
                @pl.when(g < NG - 1)
                def _():
                    pltpu.async_copy(
                        srcs_hbm.at[pl.ds(start + (g + 1) * G, G)],
                        idx_src.at[s_nxt], sem_is)
                    pltpu.async_copy(
                        dsts_hbm.at[pl.ds(start + (g + 1) * G, G)],
                        idx_dst.at[s_nxt], sem_id)
            else:
                pltpu.make_async_copy(
                    rows.at[1 - b], acc_sh.at[idx_dst.at[s_cur, 0]],
                    sem_s[1 - b]).wait()
            if jj < G - 1:
                pltpu.async_copy(h_hbm.at[idx_src.at[s_cur, jj + 1]],
                                 rows.at[1 - b], sem_g[1 - b])
            else:
                @pl.when(g < NG - 1)
                def _():
                    pltpu.make_async_copy(
                        srcs_hbm.at[pl.ds(0, G)], idx_src.at[s_nxt],
                        sem_is).wait()
                    pltpu.make_async_copy(
                        dsts_hbm.at[pl.ds(0, G)], idx_dst.at[s_nxt],
                        sem_id).wait()
                    pltpu.async_copy(h_hbm.at[idx_src.at[s_nxt, 0]],
                                     rows.at[1 - b], sem_g[1 - b])
            pltpu.async_copy(rows.at[b], acc_sh.at[idx_dst.at[s_cur, jj]],
                             sem_s[b], add=True)
        return carry
    lax.fori_loop(0, NG, _group, 0)

    pltpu.make_async_copy(rows.at[1], acc_sh.at[idx_dst.at[0, 0]],
                          sem_s[1]).wait()

    plsc.subcore_barrier()

    pltpu.sync_copy(acc_sh.at[pl.ds(sid * RPT, RPT)],
                    acc_hbm.at[cid, pl.ds(sid * RPT, RPT)])


@functools.lru_cache(maxsize=None)
def _make_seg_kernel():
    return pl.kernel(
        _seg_body,
        out_type=jax.ShapeDtypeStruct((NC, NP, D), jnp.float32),
        mesh=_mesh(),
        scratch_types=[
            pltpu.VMEM((2, G, C), jnp.int32),
            pltpu.VMEM((2, G, C), jnp.int32),
            pltpu.VMEM((2, C, D), jnp.float32),
            pltpu.VMEM_SHARED((NP, D), jnp.float32),
            pltpu.SemaphoreType.DMA,
            pltpu.SemaphoreType.DMA,
            pltpu.SemaphoreType.DMA,
            pltpu.SemaphoreType.DMA,
            pltpu.SemaphoreType.DMA,
            pltpu.SemaphoreType.DMA,
        ],
    )


def _cnt_body(dsts_hbm, cnt_hbm, dst_buf, ones_buf, cnt_sh):
    cid = lax.axis_index("c")
    sid = lax.axis_index("s")
    wid = sid * NC + cid

    zv = jnp.zeros((16,), jnp.float32)
    ov = jnp.ones((16,), jnp.float32)

    def _fill(val):
        def _row(r, carry):
            for k in range(CW // 16):
                ones_buf[r, pl.ds(k * 16, 16)] = val
            return carry
        lax.fori_loop(0, C, _row, 0)

    _fill(zv)
    for k in range(RPT // C):
        pltpu.sync_copy(ones_buf, cnt_sh.at[pl.ds(sid * RPT + k * C, C)])
    _fill(ov)

    plsc.subcore_barrier()

    jb = JT // NW
    def _group(g, carry):
        pltpu.sync_copy(dsts_hbm.at[pl.ds(wid * jb + g * G, G)], dst_buf)
        for jj in range(G):
            pltpu.sync_copy(ones_buf, cnt_sh.at[dst_buf.at[jj]], add=True)
        return carry
    lax.fori_loop(0, jb // G, _group, 0)

    plsc.subcore_barrier()

    pltpu.sync_copy(cnt_sh.at[pl.ds(sid * RPT, RPT)],
                    cnt_hbm.at[cid, pl.ds(sid * RPT, RPT)])


@functools.lru_cache(maxsize=None)
def _make_cnt_kernel():
    return pl.kernel(
        _cnt_body,
        out_type=jax.ShapeDtypeStruct((NC, NP, CW), jnp.float32),
        mesh=_mesh(),
        scratch_types=[
            pltpu.VMEM((G, C), jnp.int32),
            pltpu.VMEM((C, CW), jnp.float32),
            pltpu.VMEM_SHARED((NP, CW), jnp.float32),
        ],
    )


def _tc_body(apply_elu, p0, p1, c0, c1, h, wl, bl, wr, out):
    cnt = c0[:, 0:1] + c1[:, 0:1]
    inv = 1.0 / jnp.maximum(cnt, 1.0)
    agg = (p0[...] + p1[...]) * inv
    y = (jnp.dot(agg, wl[...], preferred_element_type=jnp.float32)
         + jnp.dot(h[...], wr[...], preferred_element_type=jnp.float32)
         + bl[...])
    if apply_elu:
        y = jnp.where(y > 0.0, y, jnp.exp(jnp.minimum(y, 0.0)) - 1.0)
    out[...] = y


def _tc_layer(apply_elu, p0, p1, c0, c1, h, wl, bl, wr):
    B = 1000
    grid = (N // B,)
    row_spec = pl.BlockSpec((B, D), lambda b: (b, 0))
    cnt_spec = pl.BlockSpec((B, CW), lambda b: (b, 0))
    w_spec = pl.BlockSpec((D, D), lambda b: (0, 0))
    b_spec = pl.BlockSpec((1, D), lambda b: (0, 0))
    return pl.pallas_call(
        functools.partial(_tc_body, apply_elu),
        grid=grid,
        in_specs=[row_spec, row_spec, cnt_spec, cnt_spec, row_spec,
                  w_spec, b_spec, w_spec],
        out_specs=row_spec,
        out_shape=jax.ShapeDtypeStruct((N, D), jnp.float32),
    )(p0, p1, c0, c1, h, wl, bl, wr)


def kernel(x, edge_index, Wl0, bl0, Wr0, Wl1, bl1, Wr1, Wl2, bl2, Wr2):
    src = edge_index[0].astype(jnp.int32)
    dst = edge_index[1].astype(jnp.int32)
    pad = EP - E
    src = jnp.concatenate([src, jnp.zeros((pad,), jnp.int32)]).reshape(JT, C)
    dst = jnp.concatenate([dst, jnp.full((pad,), N, jnp.int32)]).reshape(JT, C)

    seg = _make_seg_kernel()
    cnt = _make_cnt_kernel()(dst)
    c0, c1 = cnt[0, :N], cnt[1, :N]
    bl0 = bl0.reshape(1, D)
    bl1 = bl1.reshape(1, D)
    bl2 = bl2.reshape(1, D)

    acc = seg(x, src, dst)
    h = _tc_layer(True, acc[0, :N], acc[1, :N], c0, c1, x, Wl0, bl0, Wr0)
    acc = seg(h, src, dst)
    h = _tc_layer(True, acc[0, :N], acc[1, :N], c0, c1, h, Wl1, bl1, Wr1)
    acc = seg(h, src, dst)
    out = _tc_layer(False, acc[0, :N], acc[1, :N], c0, c1, h, Wl2, bl2, Wr2)
    return out


# speedup vs baseline: 1.9019x; 1.9019x over previous
import functools

import jax
import jax.numpy as jnp
from jax import lax
from jax.experimental import pallas as pl
from jax.experimental.pallas import tpu as pltpu
from jax.experimental.pallas import tpu_sc as plsc

N = 10000
E = 320000
D = 128
NC = 2
NS = 16
NW = NC * NS
C = 128
G = 4
JT = 2560
EP = C * JT
J0 = 156
J1 = 4
NP = 10240
RPT = NP // NS
CW = 128


def _mesh():
    return plsc.VectorSubcoreMesh(core_axis_name="c", subcore_axis_name="s",
                                  num_cores=NC, num_subcores=NS)


def _seg_body(h_hbm, srcs_hbm, dsts_hbm, acc_hbm,
              idx_src, idx_dst, rows, acc_sh,
              sem_g0, sem_g1, sem_s0, sem_s1, sem_is, sem_id):
    cid = lax.axis_index("c")
    sid = lax.axis_index("s")
    sem_g = (sem_g0, sem_g1)
    sem_s = (sem_s0, sem_s1)
    is0 = cid == 0
    start = jnp.where(is0, sid * J0, NS * J0 + sid * J1)
    NG = jnp.where(is0, J0 // G, J1 // G)

    zv = jnp.zeros((16,), jnp.float32)

    def _zrow(r, carry):
        for k in range(D // 16):
            rows[0, r, pl.ds(k * 16, 16)] = zv
        return carry
    lax.fori_loop(0, C, _zrow, 0)
    for k in range(RPT // C):
        pltpu.sync_copy(rows.at[0], acc_sh.at[pl.ds(sid * RPT + k * C, C)])

    pltpu.sync_copy(srcs_hbm.at[pl.ds(start, G)], idx_src.at[0])
    pltpu.sync_copy(dsts_hbm.at[pl.ds(start, G)], idx_dst.at[0])
    pltpu.async_copy(h_hbm.at[idx_src.at[0, 0]], rows.at[0], sem_g[0])

    plsc.subcore_barrier()

    def _group(g, carry):
        s_cur = g % 2
        s_nxt = (g + 1) % 2
        for jj in range(G):
            b = jj % 2
            pltpu.make_async_copy(h_hbm.at[idx_src.at[s_cur, jj]],
                                  rows.at[b], sem_g[b]).wait()
            if jj == 0:
                @pl.when(g >= 1)
                def _():
                    pltpu.make_async_copy(
                        rows.at[1 - b], acc_sh.at[idx_dst.at[s_cur, 0]],
                        sem_s[1 - b]).wait()
                @pl.when(g < NG - 1)
                def _():
                    pltpu.async_copy(
                        srcs_hbm.at[pl.ds(start + (g + 1) * G, G)],
                        idx_src.at[s_nxt], sem_is)
                    pltpu.async_copy(
                        dsts_hbm.at[pl.ds(start + (g + 1) * G, G)],
                        idx_dst.at[s_nxt], sem_id)
            else:
                pltpu.make_async_copy(
                    rows.at[1 - b], acc_sh.at[idx_dst.at[s_cur, 0]],
                    sem_s[1 - b]).wait()
            if jj < G - 1:
                pltpu.async_copy(h_hbm.at[idx_src.at[s_cur, jj + 1]],
                                 rows.at[1 - b], sem_g[1 - b])
            else:
                @pl.when(g < NG - 1)
                def _():
                    pltpu.make_async_copy(
                        srcs_hbm.at[pl.ds(0, G)], idx_src.at[s_nxt],
                        sem_is).wait()
                    pltpu.make_async_copy(
                        dsts_hbm.at[pl.ds(0, G)], idx_dst.at[s_nxt],
                        sem_id).wait()
                    pltpu.async_copy(h_hbm.at[idx_src.at[s_nxt, 0]],
                                     rows.at[1 - b], sem_g[1 - b])
            pltpu.async_copy(rows.at[b], acc_sh.at[idx_dst.at[s_cur, jj]],
                             sem_s[b], add=True)
        return carry
    lax.fori_loop(0, NG, _group, 0)

    pltpu.make_async_copy(rows.at[1], acc_sh.at[idx_dst.at[0, 0]],
                          sem_s[1]).wait()

    plsc.subcore_barrier()

    pltpu.sync_copy(acc_sh.at[pl.ds(sid * RPT, RPT)],
                    acc_hbm.at[cid, pl.ds(sid * RPT, RPT)])


@functools.lru_cache(maxsize=None)
def _make_seg_kernel():
    return pl.kernel(
        _seg_body,
        out_type=jax.ShapeDtypeStruct((NC, NP, D), jnp.float32),
        mesh=_mesh(),
        compiler_params=pltpu.CompilerParams(use_tc_tiling_on_sc=False),
        scratch_types=[
            pltpu.VMEM((2, G, C), jnp.int32),
            pltpu.VMEM((2, G, C), jnp.int32),
            pltpu.VMEM((2, C, D), jnp.float32),
            pltpu.VMEM_SHARED((NP, D), jnp.float32),
            pltpu.SemaphoreType.DMA,
            pltpu.SemaphoreType.DMA,
            pltpu.SemaphoreType.DMA,
            pltpu.SemaphoreType.DMA,
            pltpu.SemaphoreType.DMA,
            pltpu.SemaphoreType.DMA,
        ],
    )


DH = 64


def _seg2_body(h2_hbm, srcs_hbm, dsts_hbm, acc_hbm,
               idx_src, idx_dst, rows, table_sh, acc_sh,
               sem_g0, sem_g1, sem_s0, sem_s1, sem_is, sem_id):
    cid = lax.axis_index("c")
    sid = lax.axis_index("s")
    wid = sid * NC + cid
    sem_g = (sem_g0, sem_g1)
    sem_s = (sem_s0, sem_s1)
    jb = JT // NW
    start = wid * jb
    NG = jb // G

    zv = jnp.zeros((16,), jnp.float32)

    for half in range(2):
        pltpu.sync_copy(h2_hbm.at[half, pl.ds(sid * RPT, RPT)],
                        table_sh.at[pl.ds(sid * RPT, RPT)])
        def _zrow(r, carry):
            for k in range(DH // 16):
                rows[0, r, pl.ds(k * 16, 16)] = zv
            return carry
        lax.fori_loop(0, C, _zrow, 0)
        for k in range(RPT // C):
            pltpu.sync_copy(rows.at[0],
                            acc_sh.at[pl.ds(sid * RPT + k * C, C)])

        pltpu.sync_copy(srcs_hbm.at[pl.ds(start, G)], idx_src.at[0])
        pltpu.sync_copy(dsts_hbm.at[pl.ds(start, G)], idx_dst.at[0])

        plsc.subcore_barrier()

        pltpu.async_copy(table_sh.at[idx_src.at[0, 0]], rows.at[0], sem_g[0])

        def _group(g, carry):
            s_cur = g % 2
            s_nxt = (g + 1) % 2
            for jj in range(G):
                b = jj % 2
                pltpu.make_async_copy(table_sh.at[idx_src.at[s_cur, jj]],
                                      rows.at[b], sem_g[b]).wait()
                if jj == 0:
                    @pl.when(g >= 1)
                    def _():
                        pltpu.make_async_copy(
                            rows.at[1 - b], acc_sh.at[idx_dst.at[s_cur, 0]],
                            sem_s[1 - b]).wait()
                    @pl.when(g < NG - 1)
                    def _():
                        pltpu.async_copy(
                            srcs_hbm.at[pl.ds(start + (g + 1) * G, G)],
                            idx_src.at[s_nxt], sem_is)
                        pltpu.async_copy(
                            dsts_hbm.at[pl.ds(start + (g + 1) * G, G)],
                            idx_dst.at[s_nxt], sem_id)
                else:
                    pltpu.make_async_copy(
                        rows.at[1 - b], acc_sh.at[idx_dst.at[s_cur, 0]],
                        sem_s[1 - b]).wait()
                if jj < G - 1:
                    pltpu.async_copy(table_sh.at[idx_src.at[s_cur, jj + 1]],
                                     rows.at[1 - b], sem_g[1 - b])
                else:
                    @pl.when(g < NG - 1)
                    def _():
                        pltpu.make_async_copy(
                            srcs_hbm.at[pl.ds(0, G)], idx_src.at[s_nxt],
                            sem_is).wait()
                        pltpu.make_async_copy(
                            dsts_hbm.at[pl.ds(0, G)], idx_dst.at[s_nxt],
                            sem_id).wait()
                        pltpu.async_copy(table_sh.at[idx_src.at[s_nxt, 0]],
                                         rows.at[1 - b], sem_g[1 - b])
                pltpu.async_copy(rows.at[b], acc_sh.at[idx_dst.at[s_cur, jj]],
                                 sem_s[b], add=True)
            return carry
        lax.fori_loop(0, NG, _group, 0)

        pltpu.make_async_copy(rows.at[1], acc_sh.at[idx_dst.at[0, 0]],
                              sem_s[1]).wait()

        plsc.subcore_barrier()

        pltpu.sync_copy(acc_sh.at[pl.ds(sid * RPT, RPT)],
                        acc_hbm.at[cid, half, pl.ds(sid * RPT, RPT)])


@functools.lru_cache(maxsize=None)
def _make_seg2_kernel():
    return pl.kernel(
        _seg2_body,
        out_type=jax.ShapeDtypeStruct((NC, 2, NP, DH), jnp.float32),
        mesh=_mesh(),
        compiler_params=pltpu.CompilerParams(use_tc_tiling_on_sc=False),
        scratch_types=[
            pltpu.VMEM((2, G, C), jnp.int32),
            pltpu.VMEM((2, G, C), jnp.int32),
            pltpu.VMEM((2, C, DH), jnp.float32),
            pltpu.VMEM_SHARED((NP, DH), jnp.float32),
            pltpu.VMEM_SHARED((NP, DH), jnp.float32),
            pltpu.SemaphoreType.DMA,
            pltpu.SemaphoreType.DMA,
            pltpu.SemaphoreType.DMA,
            pltpu.SemaphoreType.DMA,
            pltpu.SemaphoreType.DMA,
            pltpu.SemaphoreType.DMA,
        ],
    )


def _cnt_body(dsts_hbm, cnt_hbm, dst_buf, ones_buf, cnt_sh):
    cid = lax.axis_index("c")
    sid = lax.axis_index("s")
    wid = sid * NC + cid

    zv = jnp.zeros((16,), jnp.float32)
    ov = jnp.ones((16,), jnp.float32)

    def _fill(val):
        def _row(r, carry):
            for k in range(CW // 16):
                ones_buf[r, pl.ds(k * 16, 16)] = val
            return carry
        lax.fori_loop(0, C, _row, 0)

    _fill(zv)
    for k in range(RPT // C):
        pltpu.sync_copy(ones_buf, cnt_sh.at[pl.ds(sid * RPT + k * C, C)])
    _fill(ov)

    plsc.subcore_barrier()

    jb = JT // NW
    def _group(g, carry):
        pltpu.sync_copy(dsts_hbm.at[pl.ds(wid * jb + g * G, G)], dst_buf)
        for jj in range(G):
            pltpu.sync_copy(ones_buf, cnt_sh.at[dst_buf.at[jj]], add=True)
        return carry
    lax.fori_loop(0, jb // G, _group, 0)

    plsc.subcore_barrier()

    pltpu.sync_copy(cnt_sh.at[pl.ds(sid * RPT, RPT)],
                    cnt_hbm.at[cid, pl.ds(sid * RPT, RPT)])


@functools.lru_cache(maxsize=None)
def _make_cnt_kernel():
    return pl.kernel(
        _cnt_body,
        out_type=jax.ShapeDtypeStruct((NC, NP, CW), jnp.float32),
        mesh=_mesh(),
        compiler_params=pltpu.CompilerParams(use_tc_tiling_on_sc=False),
        scratch_types=[
            pltpu.VMEM((G, C), jnp.int32),
            pltpu.VMEM((C, CW), jnp.float32),
            pltpu.VMEM_SHARED((NP, CW), jnp.float32),
        ],
    )


def _tc_body(apply_elu, p0, p1, c0, c1, h, wl, bl, wr, out):
    cnt = c0[:, 0:1] + c1[:, 0:1]
    inv = 1.0 / jnp.maximum(cnt, 1.0)
    agg = (p0[...] + p1[...]) * inv
    y = (jnp.dot(agg, wl[...], preferred_element_type=jnp.float32)
         + jnp.dot(h[...], wr[...], preferred_element_type=jnp.float32)
         + bl[...])
    if apply_elu:
        y = jnp.where(y > 0.0, y, jnp.exp(jnp.minimum(y, 0.0)) - 1.0)
    out[...] = y


def _tc_layer(apply_elu, p0, p1, c0, c1, h, wl, bl, wr):
    B = 1000
    grid = (N // B,)
    row_spec = pl.BlockSpec((B, D), lambda b: (b, 0))
    cnt_spec = pl.BlockSpec((B, CW), lambda b: (b, 0))
    w_spec = pl.BlockSpec((D, D), lambda b: (0, 0))
    b_spec = pl.BlockSpec((1, D), lambda b: (0, 0))
    return pl.pallas_call(
        functools.partial(_tc_body, apply_elu),
        grid=grid,
        in_specs=[row_spec, row_spec, cnt_spec, cnt_spec, row_spec,
                  w_spec, b_spec, w_spec],
        out_specs=row_spec,
        out_shape=jax.ShapeDtypeStruct((N, D), jnp.float32),
    )(p0, p1, c0, c1, h, wl, bl, wr)


def kernel(x, edge_index, Wl0, bl0, Wr0, Wl1, bl1, Wr1, Wl2, bl2, Wr2):
    src = edge_index[0].astype(jnp.int32)
    dst = edge_index[1].astype(jnp.int32)
    pad = EP - E
    src = jnp.concatenate([src, jnp.zeros((pad,), jnp.int32)]).reshape(JT, C)
    dst = jnp.concatenate([dst, jnp.full((pad,), N, jnp.int32)]).reshape(JT, C)

    seg = _make_seg2_kernel()
    zpad = jnp.zeros((2, NP - N, DH), jnp.float32)

    def agg(h):
        h2 = jnp.concatenate(
            [jnp.stack([h[:, :DH], h[:, DH:]]), zpad], axis=1)
        a = seg(h2, src, dst)
        p0 = jnp.concatenate([a[0, 0, :N], a[0, 1, :N]], axis=1)
        p1 = jnp.concatenate([a[1, 0, :N], a[1, 1, :N]], axis=1)
        return p0, p1

    cnt = _make_cnt_kernel()(dst)
    c0, c1 = cnt[0, :N], cnt[1, :N]
    bl0 = bl0.reshape(1, D)
    bl1 = bl1.reshape(1, D)
    bl2 = bl2.reshape(1, D)

    p0, p1 = agg(x)
    h = _tc_layer(True, p0, p1, c0, c1, x, Wl0, bl0, Wr0)
    p0, p1 = agg(h)
    h = _tc_layer(True, p0, p1, c0, c1, h, Wl1, bl1, Wr1)
    p0, p1 = agg(h)
    out = _tc_layer(False, p0, p1, c0, c1, h, Wl2, bl2, Wr2)
    return out
